# Initial kernel scaffold; baseline (speedup 1.0000x reference)
import jax
import jax.numpy as jnp
from jax.experimental import pallas as pl


def kernel(index_list):
    raise NotImplementedError("write your pallas kernel here")


# speedup vs baseline: 1.0104x; 1.0104x over previous
import functools

import jax
import jax.numpy as jnp
from jax import lax
from jax.experimental import pallas as pl
from jax.experimental.pallas import tpu as pltpu
from jax.experimental.pallas import tpu_sc as plsc

_BATCH = 4096
_NF = 26
_NV = 100
_D = _NF * _NV
_NC = 2
_NS = 16
_NW = _NC * _NS
_RPW = _BATCH // _NW
_CHUNK = 32
_NCH = _RPW // _CHUNK
_CW = _CHUNK * _D
_IDXW = _CHUNK * _NF


def _sc_body(idx_hbm, out_hbm, buf, idx_v, loc_v):
    wid = lax.axis_index("s") * _NC + lax.axis_index("c")
    base = wid * _RPW

    zeros = jnp.zeros((16,), jnp.int32)
    ones = jnp.ones((16,), jnp.int32)
    lane = lax.iota(jnp.int32, 16)

    def zero_body(i, carry):
        buf[pl.ds(i * 16, 16)] = zeros
        return carry

    lax.fori_loop(0, _CW // 16, zero_body, 0)

    for c in range(_NCH):
        row0 = base + c * _CHUNK
        pltpu.sync_copy(idx_hbm.at[pl.ds(row0 * _NF, _IDXW)], idx_v)
        for j in range(_IDXW // 16):
            k = lane + (j * 16)
            r = lax.div(k, _NF)
            f = k - r * _NF
            vals = idx_v[pl.ds(j * 16, 16)]
            loc = r * _D + f * _NV + vals
            loc_v[pl.ds(j * 16, 16)] = loc
            plsc.store_scatter(buf, [loc], ones)
        pltpu.sync_copy(buf, out_hbm.at[pl.ds(row0 * _D, _CW)])
        if c != _NCH - 1:
            for j in range(_IDXW // 16):
                loc = loc_v[pl.ds(j * 16, 16)]
                plsc.store_scatter(buf, [loc], zeros)


@functools.partial(
    pl.kernel,
    out_type=jax.ShapeDtypeStruct((_BATCH * _D,), jnp.int32),
    mesh=plsc.VectorSubcoreMesh(
        core_axis_name="c", subcore_axis_name="s",
        num_cores=_NC, num_subcores=_NS,
    ),
    scratch_types=[
        pltpu.VMEM((_CW,), jnp.int32),
        pltpu.VMEM((_IDXW,), jnp.int32),
        pltpu.VMEM((_IDXW,), jnp.int32),
    ],
    compiler_params=pltpu.CompilerParams(needs_layout_passes=False),
)
def _sc_multi_one_hot(idx_hbm, out_hbm, buf, idx_v, loc_v):
    _sc_body(idx_hbm, out_hbm, buf, idx_v, loc_v)


@jax.jit
def kernel(index_list):
    flat = index_list.reshape(-1)
    out = _sc_multi_one_hot(flat)
    return out.reshape(_BATCH, _D)


# speedup vs baseline: 1.4117x; 1.3972x over previous
import functools

import jax
import jax.numpy as jnp
from jax import lax
from jax.experimental import pallas as pl
from jax.experimental.pallas import tpu as pltpu
from jax.experimental.pallas import tpu_sc as plsc

_BATCH = 4096
_NF = 26
_NV = 100
_D = _NF * _NV
_NC = 2
_NS = 16
_NW = _NC * _NS
_RPW = _BATCH // _NW
_CHUNK = 32
_NCH = _RPW // _CHUNK
_IDXW = _CHUNK * _NF


def _sc_body(idx_hbm, out_hbm, buf, idx_v, loc_v):
    wid = lax.axis_index("s") * _NC + lax.axis_index("c")
    base = wid * _RPW

    zeros = jnp.zeros((16,), jnp.int32)
    ones = jnp.ones((16,), jnp.int32)
    lane = lax.iota(jnp.int32, 16)

    def zero_row(r, carry):
        def zero_col(j, carry2):
            buf[r, pl.ds(j * 16, 16)] = zeros
            return carry2
        lax.fori_loop(0, _D // 16, zero_col, 0)
        buf[r, pl.ds(_D - 16, 16)] = zeros
        return carry

    lax.fori_loop(0, _CHUNK, zero_row, 0)

    for c in range(_NCH):
        row0 = base + c * _CHUNK
        pltpu.sync_copy(idx_hbm.at[pl.ds(row0 * _NF, _IDXW)], idx_v)
        for j in range(_IDXW // 16):
            k = lane + (j * 16)
            r = lax.div(k, _NF)
            f = k - r * _NF
            vals = idx_v[pl.ds(j * 16, 16)]
            cp = f * _NV + vals
            loc_v[pl.ds(j * 16, 16)] = cp
            plsc.store_scatter(buf, [r, cp], ones)
        pltpu.sync_copy(buf, out_hbm.at[pl.ds(row0, _CHUNK)])
        if c != _NCH - 1:
            for j in range(_IDXW // 16):
                k = lane + (j * 16)
                r = lax.div(k, _NF)
                cp = loc_v[pl.ds(j * 16, 16)]
                plsc.store_scatter(buf, [r, cp], zeros)


@functools.partial(
    pl.kernel,
    out_type=jax.ShapeDtypeStruct((_BATCH, _D), jnp.int32),
    mesh=plsc.VectorSubcoreMesh(
        core_axis_name="c", subcore_axis_name="s",
        num_cores=_NC, num_subcores=_NS,
    ),
    scratch_types=[
        pltpu.VMEM((_CHUNK, _D), jnp.int32),
        pltpu.VMEM((_IDXW,), jnp.int32),
        pltpu.VMEM((_IDXW,), jnp.int32),
    ],
    compiler_params=pltpu.CompilerParams(needs_layout_passes=False),
)
def _sc_multi_one_hot(idx_hbm, out_hbm, buf, idx_v, loc_v):
    _sc_body(idx_hbm, out_hbm, buf, idx_v, loc_v)


@jax.jit
def kernel(index_list):
    flat = index_list.reshape(-1)
    return _sc_multi_one_hot(flat)


# speedup vs baseline: 1.4183x; 1.0046x over previous
import functools

import jax
import jax.numpy as jnp
from jax import lax
from jax.experimental import pallas as pl
from jax.experimental.pallas import tpu as pltpu
from jax.experimental.pallas import tpu_sc as plsc

_BATCH = 4096
_NF = 26
_NV = 100
_D = _NF * _NV
_NC = 2
_NS = 16
_NW = _NC * _NS
_RPW = _BATCH // _NW
_CHUNK = 32
_NCH = _RPW // _CHUNK
_IDXW = _CHUNK * _NF


def _sc_body(idx_hbm, out_hbm, buf, idx_v, loc_v):
    wid = lax.axis_index("s") * _NC + lax.axis_index("c")
    base = wid * _RPW

    zeros = jnp.zeros((16,), jnp.int32)
    ones = jnp.ones((16,), jnp.int32)
    lane = lax.iota(jnp.int32, 16)

    def zero_row(r, carry):
        def zero_col(j, carry2):
            buf[r, pl.ds(j * 16, 16)] = zeros
            return carry2
        lax.fori_loop(0, _D // 16, zero_col, 0)
        buf[r, pl.ds(_D - 16, 16)] = zeros
        return carry

    lax.fori_loop(0, _CHUNK, zero_row, 0)

    for c in range(_NCH):
        row0 = base + c * _CHUNK
        pltpu.sync_copy(idx_hbm.at[pl.ds(row0 * _NF, _IDXW)], idx_v)
        for j in range(_IDXW // 16):
            k = lane + (j * 16)
            r = lax.div(k, _NF)
            f = k - r * _NF
            vals = idx_v[pl.ds(j * 16, 16)]
            cp = f * _NV + vals
            loc_v[pl.ds(j * 16, 16)] = cp
            plsc.store_scatter(buf, [r, cp], ones)
        pltpu.sync_copy(buf, out_hbm.at[pl.ds(row0, _CHUNK)])
        if c != _NCH - 1:
            for j in range(_IDXW // 16):
                k = lane + (j * 16)
                r = lax.div(k, _NF)
                cp = loc_v[pl.ds(j * 16, 16)]
                plsc.store_scatter(buf, [r, cp], zeros)


@functools.partial(
    pl.kernel,
    out_type=jax.ShapeDtypeStruct((_BATCH, _D), jnp.int32),
    mesh=plsc.VectorSubcoreMesh(
        core_axis_name="c", subcore_axis_name="s",
        num_cores=_NC, num_subcores=_NS,
    ),
    scratch_types=[
        pltpu.VMEM((_CHUNK, _D), jnp.int32),
        pltpu.VMEM((_IDXW,), jnp.int32),
        pltpu.VMEM((_IDXW,), jnp.int32),
    ],
    compiler_params=pltpu.CompilerParams(
        needs_layout_passes=False, use_tc_tiling_on_sc=True),
)
def _sc_multi_one_hot(idx_hbm, out_hbm, buf, idx_v, loc_v):
    _sc_body(idx_hbm, out_hbm, buf, idx_v, loc_v)


@jax.jit
def kernel(index_list):
    flat = index_list.reshape(-1)
    return _sc_multi_one_hot(flat)


# speedup vs baseline: 2.7421x; 1.9334x over previous
import functools

import jax
import jax.numpy as jnp
from jax import lax
from jax.experimental import pallas as pl
from jax.experimental.pallas import tpu as pltpu
from jax.experimental.pallas import tpu_sc as plsc

_BATCH = 4096
_NF = 26
_NV = 100
_D = _NF * _NV
_NC = 2
_NS = 16
_CROWS = 2 * _NV
_CB = 128
_NCH = _NF // 2


def _sc_body(idx_hbm, out_hbm, buf, idx_v):
    wid = lax.axis_index("s") * _NC + lax.axis_index("c")
    c0 = wid * _CB

    zeros = jnp.zeros((16,), jnp.int32)
    ones = jnp.ones((16,), jnp.int32)
    lane = lax.iota(jnp.int32, 16)

    def zero_body(i, carry):
        for rr in range(4):
            for s in range(8):
                buf[i * 4 + rr, pl.ds(s * 16, 16)] = zeros
        return carry

    lax.fori_loop(0, _CROWS // 4, zero_body, 0)

    for p in range(_NCH):
        for fi in range(2):
            pltpu.sync_copy(
                idx_hbm.at[pl.ds((2 * p + fi) * _BATCH + c0, _CB)],
                idx_v.at[pl.ds(fi * _CB, _CB)],
            )
        for fi in range(2):
            for g in range(8):
                vals = idx_v[pl.ds(fi * _CB + g * 16, 16)]
                plsc.store_scatter(buf, [vals + (fi * _NV), lane + (g * 16)], ones)
        pltpu.sync_copy(buf, out_hbm.at[pl.ds(p * _CROWS, _CROWS), pl.ds(c0, _CB)])
        for fi in range(2):
            for g in range(8):
                vals = idx_v[pl.ds(fi * _CB + g * 16, 16)]
                plsc.store_scatter(buf, [vals + (fi * _NV), lane + (g * 16)], zeros)


@functools.partial(
    pl.kernel,
    out_type=jax.ShapeDtypeStruct((_D, _BATCH), jnp.int32),
    mesh=plsc.VectorSubcoreMesh(
        core_axis_name="c", subcore_axis_name="s",
        num_cores=_NC, num_subcores=_NS,
    ),
    scratch_types=[
        pltpu.VMEM((_CROWS, _CB), jnp.int32),
        pltpu.VMEM((2 * _CB,), jnp.int32),
    ],
    compiler_params=pltpu.CompilerParams(needs_layout_passes=False),
)
def _sc_multi_one_hot(idx_hbm, out_hbm, buf, idx_v):
    _sc_body(idx_hbm, out_hbm, buf, idx_v)


@jax.jit
def kernel(index_list):
    flat_t = index_list.T.reshape(-1)
    return _sc_multi_one_hot(flat_t).T


# speedup vs baseline: 3.8127x; 1.3904x over previous
import functools

import jax
import jax.numpy as jnp
from jax import lax
from jax.experimental import pallas as pl
from jax.experimental.pallas import tpu as pltpu
from jax.experimental.pallas import tpu_sc as plsc

_BATCH = 4096
_NF = 26
_NV = 100
_D = _NF * _NV
_NC = 2
_NS = 16
_CROWS = 2 * _NV
_CB = 128
_NCH = _NF // 2


def _sc_body(idx_hbm, out_hbm, buf0, buf1, idx_v, sem0, sem1):
    wid = lax.axis_index("s") * _NC + lax.axis_index("c")
    c0 = wid * _CB

    zeros = jnp.zeros((16,), jnp.int32)
    ones = jnp.ones((16,), jnp.int32)
    lane = lax.iota(jnp.int32, 16)

    pltpu.sync_copy(idx_hbm.at[:, pl.ds(c0, _CB)], idx_v)

    def zero_body(i, carry):
        for rr in range(4):
            for s in range(8):
                buf0[i * 4 + rr, pl.ds(s * 16, 16)] = zeros
                buf1[i * 4 + rr, pl.ds(s * 16, 16)] = zeros
        return carry

    lax.fori_loop(0, _CROWS // 4, zero_body, 0)

    bufs = (buf0, buf1)
    sems = (sem0, sem1)
    descs = [None, None]
    for p in range(_NCH):
        s = p % 2
        buf = bufs[s]
        if descs[s] is not None:
            descs[s].wait()
            for fi in range(2):
                for g in range(8):
                    vals = idx_v[2 * (p - 2) + fi, pl.ds(g * 16, 16)]
                    plsc.store_scatter(
                        buf, [vals + (fi * _NV), lane + (g * 16)], zeros)
        for fi in range(2):
            for g in range(8):
                vals = idx_v[2 * p + fi, pl.ds(g * 16, 16)]
                plsc.store_scatter(
                    buf, [vals + (fi * _NV), lane + (g * 16)], ones)
        descs[s] = pltpu.async_copy(
            buf, out_hbm.at[pl.ds(p * _CROWS, _CROWS), pl.ds(c0, _CB)], sems[s])
    descs[(_NCH - 1) % 2].wait()
    descs[_NCH % 2].wait()


@functools.partial(
    pl.kernel,
    out_type=jax.ShapeDtypeStruct((_D, _BATCH), jnp.int32),
    mesh=plsc.VectorSubcoreMesh(
        core_axis_name="c", subcore_axis_name="s",
        num_cores=_NC, num_subcores=_NS,
    ),
    scratch_types=[
        pltpu.VMEM((_CROWS, _CB), jnp.int32),
        pltpu.VMEM((_CROWS, _CB), jnp.int32),
        pltpu.VMEM((_NF, _CB), jnp.int32),
        pltpu.SemaphoreType.DMA,
        pltpu.SemaphoreType.DMA,
    ],
    compiler_params=pltpu.CompilerParams(needs_layout_passes=False),
)
def _sc_multi_one_hot(idx_hbm, out_hbm, buf0, buf1, idx_v, sem0, sem1):
    _sc_body(idx_hbm, out_hbm, buf0, buf1, idx_v, sem0, sem1)


@jax.jit
def kernel(index_list):
    return _sc_multi_one_hot(index_list.T).T


# speedup vs baseline: 4.0546x; 1.0635x over previous
import functools

import jax
import jax.numpy as jnp
from jax import lax
from jax.experimental import pallas as pl
from jax.experimental.pallas import tpu as pltpu
from jax.experimental.pallas import tpu_sc as plsc

_BATCH = 4096
_NF = 26
_NV = 100
_D = _NF * _NV
_NC = 2
_NS = 16
_CROWS = 2 * _NV
_CB = 128
_NCH = _NF // 2


def _sc_body(idx_hbm, out_hbm, buf0, buf1, idx_v, sem0, sem1):
    wid = lax.axis_index("s") * _NC + lax.axis_index("c")
    c0 = wid * _CB

    zeros = jnp.zeros((16,), jnp.int32)
    ones = jnp.ones((16,), jnp.int32)
    lane = lax.iota(jnp.int32, 16)

    pltpu.sync_copy(idx_hbm.at[:, pl.ds(c0, _CB)], idx_v)

    def zero_body(i, carry):
        for rr in range(4):
            for s in range(8):
                buf0[i * 4 + rr, pl.ds(s * 16, 16)] = zeros
                buf1[i * 4 + rr, pl.ds(s * 16, 16)] = zeros
        return carry

    lax.fori_loop(0, _CROWS // 4, zero_body, 0)

    bufs = (buf0, buf1)
    sems = (sem0, sem1)

    def win(p):
        r = pl.multiple_of(p * _CROWS, _CROWS)
        return out_hbm.at[pl.ds(r, _CROWS), pl.ds(c0, _CB)]

    def scat(buf, p, x):
        for fi in range(2):
            for g in range(8):
                vals = idx_v[2 * p + fi, pl.ds(g * 16, 16)]
                plsc.store_scatter(
                    buf, [vals + (fi * _NV), lane + (g * 16)], x)

    def step(buf, sem, p):
        pltpu.make_async_copy(buf, win(p - 2), sem).wait()
        scat(buf, p - 2, zeros)
        scat(buf, p, ones)
        pltpu.async_copy(buf, win(p), sem)

    scat(buf0, 0, ones)
    pltpu.async_copy(buf0, win(0), sem0)
    scat(buf1, 1, ones)
    pltpu.async_copy(buf1, win(1), sem1)

    def pair_body(j, carry):
        step(buf0, sem0, 2 * j + 2)
        step(buf1, sem1, 2 * j + 3)
        return carry

    lax.fori_loop(0, (_NCH - 3) // 2, pair_body, 0)

    step(buf0, sem0, _NCH - 1)
    pltpu.make_async_copy(buf1, win(_NCH - 2), sem1).wait()
    pltpu.make_async_copy(buf0, win(_NCH - 1), sem0).wait()


@functools.partial(
    pl.kernel,
    out_type=jax.ShapeDtypeStruct((_D, _BATCH), jnp.int32),
    mesh=plsc.VectorSubcoreMesh(
        core_axis_name="c", subcore_axis_name="s",
        num_cores=_NC, num_subcores=_NS,
    ),
    scratch_types=[
        pltpu.VMEM((_CROWS, _CB), jnp.int32),
        pltpu.VMEM((_CROWS, _CB), jnp.int32),
        pltpu.VMEM((_NF, _CB), jnp.int32),
        pltpu.SemaphoreType.DMA,
        pltpu.SemaphoreType.DMA,
    ],
    compiler_params=pltpu.CompilerParams(needs_layout_passes=False),
)
def _sc_multi_one_hot(idx_hbm, out_hbm, buf0, buf1, idx_v, sem0, sem1):
    _sc_body(idx_hbm, out_hbm, buf0, buf1, idx_v, sem0, sem1)


@jax.jit
def kernel(index_list):
    return _sc_multi_one_hot(index_list.T).T
